# baseline (speedup 1.0000x reference)
import functools

import jax
import jax.numpy as jnp
import numpy as np
from jax.experimental import pallas as pl
from jax.experimental.pallas import tpu as pltpu
from jax.sharding import Mesh, PartitionSpec as P

_TB = 2048


def _fused_kernel(x_ref, w_ref, b_ref, y_ref, reg_ref, *, l1, l2):
    w = w_ref[...]
    y_ref[...] = jnp.dot(x_ref[...], w, preferred_element_type=jnp.float32) + b_ref[0]
    reg_ref[...] = (l1 * jnp.sum(jnp.abs(w)) + l2 * jnp.sqrt(jnp.sum(w * w))).reshape(
        1, 1
    )


def _linreg_pallas(x, wt, bias):
    b_loc, d = x.shape
    tb = min(_TB, b_loc)
    grid = (pl.cdiv(b_loc, tb),)
    y, reg = pl.pallas_call(
        functools.partial(_fused_kernel, l1=0.01, l2=0.01),
        grid=grid,
        in_specs=[
            pl.BlockSpec((tb, d), lambda i: (i, 0)),
            pl.BlockSpec((d, 1), lambda i: (0, 0)),
            pl.BlockSpec(memory_space=pltpu.MemorySpace.SMEM),
        ],
        out_specs=[
            pl.BlockSpec((tb, 1), lambda i: (i, 0)),
            pl.BlockSpec((1, 1), lambda i: (0, 0)),
        ],
        out_shape=[
            jax.ShapeDtypeStruct((b_loc, 1), jnp.float32),
            jax.ShapeDtypeStruct((1, 1), jnp.float32),
        ],
        compiler_params=pltpu.CompilerParams(
            dimension_semantics=("parallel",),
            vmem_limit_bytes=64 * 1024 * 1024,
        ),
    )(x, wt, bias)
    return y, reg


def kernel(x, weight, bias):
    B, D = x.shape
    wt = weight.reshape(D, 1)

    devs = jax.devices()
    n = len(devs) if (len(devs) > 1 and B % len(devs) == 0) else 1
    if n == 1:
        y, reg = _linreg_pallas(x, wt, bias)
        return y, reg[0, 0]

    mesh = Mesh(np.array(devs), ("b",))
    f = jax.shard_map(
        _linreg_pallas,
        mesh=mesh,
        in_specs=(P("b", None), P(None, None), P(None)),
        out_specs=(P("b", None), P(None, None)),
        check_vma=False,
    )
    y, reg = f(x, wt, bias)
    return y, reg[0, 0]


# speedup vs baseline: 17.7558x; 17.7558x over previous
import functools

import jax
import jax.numpy as jnp
from jax.experimental import pallas as pl
from jax.experimental.pallas import tpu as pltpu

_TB = 4096


def _fused_kernel(x_ref, w_ref, b_ref, y_ref, reg_ref, *, l1, l2):
    w = w_ref[...]
    y_ref[...] = jnp.dot(x_ref[...], w, preferred_element_type=jnp.float32) + b_ref[0]
    reg_ref[...] = (l1 * jnp.sum(jnp.abs(w)) + l2 * jnp.sqrt(jnp.sum(w * w))).reshape(
        1, 1
    )


def kernel(x, weight, bias):
    B, D = x.shape
    tb = min(_TB, B)
    grid = (pl.cdiv(B, tb),)

    wt = weight.reshape(D, 1)

    y, reg = pl.pallas_call(
        functools.partial(_fused_kernel, l1=0.01, l2=0.01),
        grid=grid,
        in_specs=[
            pl.BlockSpec((tb, D), lambda i: (i, 0)),
            pl.BlockSpec((D, 1), lambda i: (0, 0)),
            pl.BlockSpec(memory_space=pltpu.MemorySpace.SMEM),
        ],
        out_specs=[
            pl.BlockSpec((tb, 1), lambda i: (i, 0)),
            pl.BlockSpec((1, 1), lambda i: (0, 0)),
        ],
        out_shape=[
            jax.ShapeDtypeStruct((B, 1), jnp.float32),
            jax.ShapeDtypeStruct((1, 1), jnp.float32),
        ],
        compiler_params=pltpu.CompilerParams(
            dimension_semantics=("parallel",),
            vmem_limit_bytes=64 * 1024 * 1024,
        ),
    )(x, wt, bias)
    return y, reg[0, 0]


# speedup vs baseline: 19.5133x; 1.0990x over previous
import functools

import jax
import jax.numpy as jnp
from jax.experimental import pallas as pl
from jax.experimental.pallas import tpu as pltpu

_TB = 4096


def _fused_kernel(x_ref, w_ref, b_ref, y_ref, reg_ref, *, l1, l2):
    w = w_ref[...]
    y_ref[...] = jnp.sum(x_ref[...] * w, axis=1, keepdims=True) + b_ref[0]
    reg_ref[...] = (l1 * jnp.sum(jnp.abs(w)) + l2 * jnp.sqrt(jnp.sum(w * w))).reshape(
        1, 1
    )


def kernel(x, weight, bias):
    B, D = x.shape
    tb = min(_TB, B)
    grid = (pl.cdiv(B, tb),)

    y, reg = pl.pallas_call(
        functools.partial(_fused_kernel, l1=0.01, l2=0.01),
        grid=grid,
        in_specs=[
            pl.BlockSpec((tb, D), lambda i: (i, 0)),
            pl.BlockSpec((1, D), lambda i: (0, 0)),
            pl.BlockSpec(memory_space=pltpu.MemorySpace.SMEM),
        ],
        out_specs=[
            pl.BlockSpec((tb, 1), lambda i: (i, 0)),
            pl.BlockSpec((1, 1), lambda i: (0, 0)),
        ],
        out_shape=[
            jax.ShapeDtypeStruct((B, 1), jnp.float32),
            jax.ShapeDtypeStruct((1, 1), jnp.float32),
        ],
        compiler_params=pltpu.CompilerParams(
            dimension_semantics=("parallel",),
            vmem_limit_bytes=64 * 1024 * 1024,
        ),
    )(x, weight, bias)
    return y, reg[0, 0]
